# baseline (speedup 1.0000x reference)
import jax
import jax.numpy as jnp
from jax.experimental import pallas as pl
from jax.experimental.pallas import tpu as pltpu

END_TOK = 49407
FILL_ROWS = 2048


def _body(plen_ref, vt_ref, emb_ref, out_ref, buf0, buf1, end_buf, bnd_buf,
          sem_end, rsem0, rsem1, wsem0, wsem1, fsem, bsem, bwsem):
    B, L, D = vt_ref.shape
    bufs = (buf0, buf1)
    rsems = (rsem0, rsem1)
    wsems = (wsem0, wsem1)

    end_dma = pltpu.make_async_copy(
        emb_ref.at[pl.ds(END_TOK - 7, 8)], end_buf.at[pl.ds(0, 8)], sem_end)
    end_dma.start()
    end_dma.wait()
    end_row = end_buf[7:8, :]
    end_buf[...] = jnp.broadcast_to(end_row, (FILL_ROWS, D))

    def c8_of(i):
        return pl.multiple_of(jnp.minimum((plen_ref[i] + 7) & ~7, L), 8)

    def f8_of(i):
        return pl.multiple_of(plen_ref[i] & ~7, 8)

    def copy_dma(b, src, dst, sem, do_start):
        f8 = f8_of(b)
        for k in range(11, 2, -1):
            size = 1 << k
            off = pl.multiple_of((f8 >> (k + 1)) << (k + 1), size * 2)
            @pl.when((f8 & size) != 0)
            def _():
                dma = pltpu.make_async_copy(
                    src.at[pl.ds(off, size)], dst.at[pl.ds(off, size)], sem)
                dma.start() if do_start else dma.wait()

    def fill_dma(b, do_start):
        c8 = c8_of(b)
        q = (L - c8) >> 3
        for k in range(8, -1, -1):
            rows = 8 << k
            off = pl.multiple_of(c8 + ((q >> (k + 1)) << (k + 1)) * 8, 8)
            @pl.when((q & (1 << k)) != 0)
            def _():
                dma = pltpu.make_async_copy(
                    end_buf.at[pl.ds(0, rows)],
                    out_ref.at[b, pl.ds(off, rows)],
                    fsem,
                )
                dma.start() if do_start else dma.wait()
        dma = pltpu.make_async_copy(
            end_buf.at[pl.ds(0, 1)], out_ref.at[b, pl.ds(L, 1)], fsem)
        dma.start() if do_start else dma.wait()

    def bnd_dma(b, writing, do_start):
        plen = plen_ref[b]
        c8 = c8_of(b)
        @pl.when(c8 > plen)
        def _():
            f8 = pl.multiple_of(c8 - 8, 8)
            if writing:
                dma = pltpu.make_async_copy(
                    bnd_buf.at[b], out_ref.at[b, pl.ds(f8, 8)], bwsem)
            else:
                dma = pltpu.make_async_copy(
                    vt_ref.at[b, pl.ds(f8, 8)], bnd_buf.at[b], bsem)
            dma.start() if do_start else dma.wait()

    def stage_a(b):
        if b >= 2:
            copy_dma(b - 2, bufs[b % 2], out_ref.at[b - 2], wsems[b % 2],
                     False)
        copy_dma(b, vt_ref.at[b], bufs[b % 2], rsems[b % 2], True)
        bnd_dma(b, False, True)

    rows8 = jax.lax.broadcasted_iota(jnp.int32, (8, 1), 0)

    def stage_b(b):
        copy_dma(b, vt_ref.at[b], bufs[b % 2], rsems[b % 2], False)
        copy_dma(b, bufs[b % 2], out_ref.at[b], wsems[b % 2], True)
        fill_dma(b, True)
        plen = plen_ref[b]
        c8 = c8_of(b)
        bnd_dma(b, False, False)
        @pl.when(c8 > plen)
        def _():
            f8 = pl.multiple_of(c8 - 8, 8)
            r = plen - f8
            bnd_buf[b] = jnp.where(rows8 < r, bnd_buf[b], end_row)
        bnd_dma(b, True, True)

    stage_a(0)
    stage_a(1)
    for b in range(B - 2):
        stage_b(b)
        stage_a(b + 2)
    stage_b(B - 2)
    stage_b(B - 1)

    copy_dma(B - 2, bufs[(B - 2) % 2], out_ref.at[B - 2], wsems[(B - 2) % 2],
             False)
    copy_dma(B - 1, bufs[(B - 1) % 2], out_ref.at[B - 1], wsems[(B - 1) % 2],
             False)
    for b in range(B):
        fill_dma(b, False)
        bnd_dma(b, True, False)


def kernel(categories, vt, emb):
    B, L = categories.shape
    D = vt.shape[-1]
    plen = jnp.sum((categories != END_TOK).astype(jnp.int32), axis=1)

    grid_spec = pltpu.PrefetchScalarGridSpec(
        num_scalar_prefetch=1,
        grid=(1,),
        in_specs=[
            pl.BlockSpec(memory_space=pl.ANY),
            pl.BlockSpec(memory_space=pl.ANY),
        ],
        out_specs=pl.BlockSpec(memory_space=pl.ANY),
        scratch_shapes=[
            pltpu.VMEM((L, D), jnp.float32),
            pltpu.VMEM((L, D), jnp.float32),
            pltpu.VMEM((FILL_ROWS, D), jnp.float32),
            pltpu.VMEM((B, 8, D), jnp.float32),
            pltpu.SemaphoreType.DMA,
            pltpu.SemaphoreType.DMA,
            pltpu.SemaphoreType.DMA,
            pltpu.SemaphoreType.DMA,
            pltpu.SemaphoreType.DMA,
            pltpu.SemaphoreType.DMA,
            pltpu.SemaphoreType.DMA,
            pltpu.SemaphoreType.DMA,
        ],
    )

    return pl.pallas_call(
        _body,
        grid_spec=grid_spec,
        out_shape=jax.ShapeDtypeStruct((B, L + 1, D), vt.dtype),
    )(plen, vt, emb)
